# baseline (speedup 1.0000x reference)
import functools

import jax
import jax.numpy as jnp
from jax import lax
from jax.experimental import pallas as pl
from jax.experimental.pallas import tpu as pltpu
from jax.experimental.pallas import tpu_sc as plsc

_M = 18432
_N = 8192
_K = 256
_MB = 256
_GRID = _M // _MB

_NW = 32
_RPW = _M // _NW
_CH = 96
_NCH = _RPW // _CH

_COMMIT = 0.25


def _dist_argmin_body(x_ref, e_ref, esq_ref, xsq_ref, idx_ref, losssum_ref,
                      cnt_ref):
    i = pl.program_id(0)
    x = x_ref[...]
    xsq = xsq_ref[...]
    cross = lax.dot_general(
        x.astype(jnp.bfloat16), e_ref[...].astype(jnp.bfloat16),
        (((1,), (1,)), ((), ())),
        preferred_element_type=jnp.float32)
    d2 = jnp.maximum(xsq - 2.0 * cross + esq_ref[...], 0.0)
    dist = jnp.sqrt(d2)
    minval = jnp.min(dist, axis=1, keepdims=True)
    iota = lax.broadcasted_iota(jnp.int32, (_MB, _N), 1)
    idx = jnp.min(jnp.where(dist == minval, iota, _N), axis=1)
    idx_ref[...] = idx[:, None]
    blk = jnp.sum(jnp.min(d2, axis=1))
    onehot = jnp.where(iota == idx[:, None], 1.0, 0.0)
    cnt = jnp.sum(onehot, axis=0, keepdims=True)

    @pl.when(i == 0)
    def _():
        losssum_ref[...] = blk.reshape(1, 1)
        cnt_ref[...] = cnt

    @pl.when(i != 0)
    def _():
        losssum_ref[...] += blk.reshape(1, 1)
        cnt_ref[...] += cnt


_dist_argmin = pl.pallas_call(
    _dist_argmin_body,
    grid=(_GRID,),
    in_specs=[
        pl.BlockSpec((_MB, _K), lambda i: (i, 0)),
        pl.BlockSpec((_N, _K), lambda i: (0, 0)),
        pl.BlockSpec((1, _N), lambda i: (0, 0)),
        pl.BlockSpec((_MB, 1), lambda i: (i, 0)),
    ],
    out_specs=[
        pl.BlockSpec((_MB, 1), lambda i: (i, 0)),
        pl.BlockSpec((1, 1), lambda i: (0, 0)),
        pl.BlockSpec((1, _N), lambda i: (0, 0)),
    ],
    out_shape=[
        jax.ShapeDtypeStruct((_M, 1), jnp.int32),
        jax.ShapeDtypeStruct((1, 1), jnp.float32),
        jax.ShapeDtypeStruct((1, _N), jnp.float32),
    ],
    compiler_params=pltpu.CompilerParams(
        dimension_semantics=("arbitrary",)),
)


def _sc_gather_body(emb_hbm, idx_hbm, quant_hbm, idx_v, rows_v, sem):
    wid = lax.axis_index("s") * 2 + lax.axis_index("c")
    base = wid * _RPW
    for c in range(_NCH):
        off = base + c * _CH
        pltpu.sync_copy(idx_hbm.at[pl.ds(off, _CH)], idx_v)
        pltpu.async_copy(emb_hbm.at[idx_v], rows_v, sem).wait()
        pltpu.sync_copy(rows_v, quant_hbm.at[pl.ds(off, _CH)])


@functools.cache
def _sc_gather():
    return functools.partial(
        pl.kernel,
        mesh=plsc.VectorSubcoreMesh(core_axis_name="c", subcore_axis_name="s"),
        out_type=jax.ShapeDtypeStruct((_M, _K), jnp.float32),
        scratch_types=[
            pltpu.VMEM((_CH,), jnp.int32),
            pltpu.VMEM((_CH, _K), jnp.float32),
            pltpu.SemaphoreType.DMA,
        ],
    )(_sc_gather_body)


def _finalize_body(counts_ref, losssum_ref, loss_ref, perp_ref):
    avg = counts_ref[...] / jnp.float32(_M)
    ent = -jnp.sum(avg * jnp.log(avg + 1e-10))
    perp_ref[...] = jnp.exp(ent).reshape(1, 1)
    m = losssum_ref[...] / jnp.float32(_M * _K)
    loss_ref[...] = m + _COMMIT * m


_finalize = pl.pallas_call(
    _finalize_body,
    out_shape=[
        jax.ShapeDtypeStruct((1, 1), jnp.float32),
        jax.ShapeDtypeStruct((1, 1), jnp.float32),
    ],
)


def kernel(inputs, embedding_weight):
    batch, seq, dim = inputs.shape
    flat = inputs.reshape(batch * seq, dim)
    esq = jnp.sum(embedding_weight ** 2, axis=1)[None, :]
    xsq = jnp.sum(flat ** 2, axis=1, keepdims=True)
    idx2d, losssum, counts = _dist_argmin(flat, embedding_weight, esq, xsq)
    idx = idx2d.reshape(_M)
    quant = _sc_gather()(embedding_weight, idx)
    loss2d, perp2d = _finalize(counts, losssum)
    return (quant.reshape(inputs.shape), loss2d.reshape(()),
            perp2d.reshape(()), idx)


# speedup vs baseline: 1.0524x; 1.0524x over previous
import functools

import jax
import jax.numpy as jnp
from jax import lax
from jax.experimental import pallas as pl
from jax.experimental.pallas import tpu as pltpu
from jax.experimental.pallas import tpu_sc as plsc

_M = 18432
_N = 8192
_K = 256
_MB = 256
_GRID = _M // _MB

_NW = 32
_RPW = _M // _NW
_CH = 96
_NCH = _RPW // _CH

_COMMIT = 0.25


def _dist_argmin_body(x_ref, e_ref, esq_ref, xsq_ref, idx_ref, losssum_ref,
                      cnt_ref):
    i = pl.program_id(0)
    x = x_ref[...]
    xsq = xsq_ref[...]
    cross = lax.dot_general(
        x.astype(jnp.bfloat16), e_ref[...].astype(jnp.bfloat16),
        (((1,), (1,)), ((), ())),
        preferred_element_type=jnp.float32)
    d2 = jnp.maximum(xsq - 2.0 * cross + esq_ref[...], 0.0)
    dist = jnp.sqrt(d2)
    minval = jnp.min(dist, axis=1, keepdims=True)
    iota = lax.broadcasted_iota(jnp.int32, (_MB, _N), 1)
    idx = jnp.min(jnp.where(dist == minval, iota, _N), axis=1)
    idx_ref[...] = idx[:, None]
    blk = jnp.sum(minval * minval)
    onehot = jnp.where(iota == idx[:, None], 1.0, 0.0)
    cnt = jnp.sum(onehot, axis=0, keepdims=True)

    @pl.when(i == 0)
    def _():
        losssum_ref[...] = blk.reshape(1, 1)
        cnt_ref[...] = cnt

    @pl.when(i != 0)
    def _():
        losssum_ref[...] += blk.reshape(1, 1)
        cnt_ref[...] += cnt


_dist_argmin = pl.pallas_call(
    _dist_argmin_body,
    grid=(_GRID,),
    in_specs=[
        pl.BlockSpec((_MB, _K), lambda i: (i, 0)),
        pl.BlockSpec((_N, _K), lambda i: (0, 0)),
        pl.BlockSpec((1, _N), lambda i: (0, 0)),
        pl.BlockSpec((_MB, 1), lambda i: (i, 0)),
    ],
    out_specs=[
        pl.BlockSpec((_MB, 1), lambda i: (i, 0)),
        pl.BlockSpec((1, 1), lambda i: (0, 0)),
        pl.BlockSpec((1, _N), lambda i: (0, 0)),
    ],
    out_shape=[
        jax.ShapeDtypeStruct((_M, 1), jnp.int32),
        jax.ShapeDtypeStruct((1, 1), jnp.float32),
        jax.ShapeDtypeStruct((1, _N), jnp.float32),
    ],
    compiler_params=pltpu.CompilerParams(
        dimension_semantics=("arbitrary",)),
)


def _sc_gather_body(emb_hbm, idx_hbm, quant_hbm, idx_v, rows_v, sem):
    wid = lax.axis_index("s") * 2 + lax.axis_index("c")
    base = wid * _RPW
    for c in range(_NCH):
        pltpu.sync_copy(idx_hbm.at[pl.ds(base + c * _CH, _CH)], idx_v)
        pltpu.async_copy(emb_hbm.at[idx_v], rows_v, sem).wait()
        pltpu.sync_copy(rows_v, quant_hbm.at[wid, pl.ds(c * _CH, _CH)])


@functools.cache
def _sc_gather():
    return functools.partial(
        pl.kernel,
        mesh=plsc.VectorSubcoreMesh(core_axis_name="c", subcore_axis_name="s"),
        out_type=jax.ShapeDtypeStruct((_NW, _RPW, _K), jnp.float32),
        scratch_types=[
            pltpu.VMEM((_CH,), jnp.int32),
            pltpu.VMEM((_CH, _K), jnp.float32),
            pltpu.SemaphoreType.DMA,
        ],
    )(_sc_gather_body)


def _finalize_body(counts_ref, losssum_ref, loss_ref, perp_ref):
    avg = counts_ref[...] / jnp.float32(_M)
    ent = -jnp.sum(avg * jnp.log(avg + 1e-10))
    perp_ref[...] = jnp.exp(ent).reshape(1, 1)
    m = losssum_ref[...] / jnp.float32(_M * _K)
    loss_ref[...] = m + _COMMIT * m


_finalize = pl.pallas_call(
    _finalize_body,
    out_shape=[
        jax.ShapeDtypeStruct((1, 1), jnp.float32),
        jax.ShapeDtypeStruct((1, 1), jnp.float32),
    ],
)


def kernel(inputs, embedding_weight):
    batch, seq, dim = inputs.shape
    flat = inputs.reshape(batch * seq, dim)
    esq = jnp.sum(embedding_weight ** 2, axis=1)[None, :]
    xsq = jnp.sum(flat ** 2, axis=1, keepdims=True)
    idx2d, losssum, counts = _dist_argmin(flat, embedding_weight, esq, xsq)
    idx = idx2d.reshape(_M)
    quant = _sc_gather()(embedding_weight, idx)
    loss2d, perp2d = _finalize(counts, losssum)
    return (quant, loss2d.reshape(()), perp2d.reshape(()), idx)


# speedup vs baseline: 1.2291x; 1.1679x over previous
import functools

import jax
import jax.numpy as jnp
from jax import lax
from jax.experimental import pallas as pl
from jax.experimental.pallas import tpu as pltpu
from jax.experimental.pallas import tpu_sc as plsc

_M = 18432
_N = 8192
_K = 256
_MB = 256
_GRID = _M // _MB

_NW = 32
_RPW = _M // _NW
_CH = 96
_NCH = _RPW // _CH

_COMMIT = 0.25


def _dist_argmin_body(x_ref, e_ref, esq_ref, xsq_ref, idx_ref, losssum_ref,
                      cnt_ref):
    i = pl.program_id(0)
    x = x_ref[...]
    xsq = xsq_ref[...]
    cross = lax.dot_general(
        x.astype(jnp.bfloat16), e_ref[...],
        (((1,), (1,)), ((), ())),
        preferred_element_type=jnp.float32)
    d2 = jnp.maximum(xsq - 2.0 * cross + esq_ref[...], 0.0)
    dist = jnp.sqrt(d2)
    minval = jnp.min(dist, axis=1, keepdims=True)
    iota = lax.broadcasted_iota(jnp.int32, (_MB, _N), 1)
    idx = jnp.min(jnp.where(dist == minval, iota, _N), axis=1)
    idx_ref[...] = idx[:, None]
    blk = jnp.sum(minval * minval)
    hi = lax.shift_right_logical(idx, 7)[:, None]
    lo = lax.bitwise_and(idx, 127)[:, None]
    iota64 = lax.broadcasted_iota(jnp.int32, (_MB, 64), 1)
    iota128 = lax.broadcasted_iota(jnp.int32, (_MB, 128), 1)
    a = jnp.where(iota64 == hi, 1.0, 0.0).astype(jnp.bfloat16)
    b = jnp.where(iota128 == lo, 1.0, 0.0).astype(jnp.bfloat16)
    cnt = lax.dot_general(a, b, (((0,), (0,)), ((), ())),
                          preferred_element_type=jnp.float32)

    @pl.when(i == 0)
    def _():
        losssum_ref[...] = blk.reshape(1, 1)
        cnt_ref[...] = cnt

    @pl.when(i != 0)
    def _():
        losssum_ref[...] += blk.reshape(1, 1)
        cnt_ref[...] += cnt


_dist_argmin = pl.pallas_call(
    _dist_argmin_body,
    grid=(_GRID,),
    in_specs=[
        pl.BlockSpec((_MB, _K), lambda i: (i, 0)),
        pl.BlockSpec((_N, _K), lambda i: (0, 0)),
        pl.BlockSpec((1, _N), lambda i: (0, 0)),
        pl.BlockSpec((_MB, 1), lambda i: (i, 0)),
    ],
    out_specs=[
        pl.BlockSpec((_MB, 1), lambda i: (i, 0)),
        pl.BlockSpec((1, 1), lambda i: (0, 0)),
        pl.BlockSpec((64, 128), lambda i: (0, 0)),
    ],
    out_shape=[
        jax.ShapeDtypeStruct((_M, 1), jnp.int32),
        jax.ShapeDtypeStruct((1, 1), jnp.float32),
        jax.ShapeDtypeStruct((64, 128), jnp.float32),
    ],
    compiler_params=pltpu.CompilerParams(
        dimension_semantics=("arbitrary",)),
)


def _sc_gather_body(emb_hbm, idx_hbm, quant_hbm, idx_v, rows_v, sem):
    wid = lax.axis_index("s") * 2 + lax.axis_index("c")
    base = wid * _RPW
    for c in range(_NCH):
        pltpu.sync_copy(idx_hbm.at[pl.ds(base + c * _CH, _CH)], idx_v)
        pltpu.async_copy(emb_hbm.at[idx_v], rows_v, sem).wait()
        pltpu.sync_copy(rows_v, quant_hbm.at[wid, pl.ds(c * _CH, _CH)])


@functools.cache
def _sc_gather():
    return functools.partial(
        pl.kernel,
        mesh=plsc.VectorSubcoreMesh(core_axis_name="c", subcore_axis_name="s"),
        out_type=jax.ShapeDtypeStruct((_NW, _RPW, _K), jnp.float32),
        scratch_types=[
            pltpu.VMEM((_CH,), jnp.int32),
            pltpu.VMEM((_CH, _K), jnp.float32),
            pltpu.SemaphoreType.DMA,
        ],
    )(_sc_gather_body)


def _finalize_body(counts_ref, losssum_ref, loss_ref, perp_ref):
    avg = counts_ref[...] / jnp.float32(_M)
    ent = -jnp.sum(avg * jnp.log(avg + 1e-10))
    perp_ref[...] = jnp.exp(ent).reshape(1, 1)
    m = losssum_ref[...] / jnp.float32(_M * _K)
    loss_ref[...] = m + _COMMIT * m


_finalize = pl.pallas_call(
    _finalize_body,
    out_shape=[
        jax.ShapeDtypeStruct((1, 1), jnp.float32),
        jax.ShapeDtypeStruct((1, 1), jnp.float32),
    ],
)


def kernel(inputs, embedding_weight):
    batch, seq, dim = inputs.shape
    flat = inputs.reshape(batch * seq, dim)
    esq = jnp.sum(embedding_weight ** 2, axis=1)[None, :]
    xsq = jnp.sum(flat ** 2, axis=1, keepdims=True)
    ebf = embedding_weight.astype(jnp.bfloat16)
    idx2d, losssum, counts = _dist_argmin(flat, ebf, esq, xsq)
    idx = idx2d.reshape(_M)
    quant = _sc_gather()(embedding_weight, idx)
    loss2d, perp2d = _finalize(counts, losssum)
    return (quant, loss2d.reshape(()), perp2d.reshape(()), idx)
